# baseline (speedup 1.0000x reference)
import functools

import jax
import jax.numpy as jnp
from jax import lax
from jax.experimental import pallas as pl
from jax.experimental.pallas import tpu as pltpu
from jax.experimental.pallas import tpu_sc as plsc

_NUM_CORES = 2
_NUM_SUBCORES = 16
_NUM_WORKERS = _NUM_CORES * _NUM_SUBCORES
_K = 64


def kernel(embeddings, token_type, modality_embedding_weight):
    seq_len = embeddings.shape[1]
    d = modality_embedding_weight.shape[1]
    rows_per_worker = seq_len // _NUM_WORKERS
    reps = rows_per_worker // _K

    idx = jnp.full((_K,), token_type, dtype=jnp.int32)

    mesh = plsc.VectorSubcoreMesh(core_axis_name="c", subcore_axis_name="s")

    @functools.partial(
        pl.kernel,
        mesh=mesh,
        out_type=jax.ShapeDtypeStruct((seq_len, d), jnp.float32),
        scratch_types=[
            pltpu.VMEM((_K,), jnp.int32),
            pltpu.VMEM((_K, d), jnp.float32),
            pltpu.SemaphoreType.DMA,
        ],
    )
    def bcast(idx_hbm, w_hbm, out_hbm, idx_v, buf_v, sem):
        wid = lax.axis_index("s") * _NUM_CORES + lax.axis_index("c")
        base = wid * rows_per_worker
        pltpu.sync_copy(idx_hbm, idx_v)
        pltpu.async_copy(w_hbm.at[idx_v], buf_v, sem).wait()
        copies = [
            pltpu.async_copy(buf_v, out_hbm.at[pl.ds(base + j * _K, _K)], sem)
            for j in range(reps)
        ]
        for c in copies:
            c.wait()

    return bcast(idx, modality_embedding_weight)


# speedup vs baseline: 2.4318x; 2.4318x over previous
import functools

import jax
import jax.numpy as jnp
from jax import lax
from jax.experimental import pallas as pl
from jax.experimental.pallas import tpu as pltpu
from jax.experimental.pallas import tpu_sc as plsc

_NUM_CORES = 2
_NUM_SUBCORES = 16
_NUM_WORKERS = _NUM_CORES * _NUM_SUBCORES
_KG = 8
_K = 8


def kernel(embeddings, token_type, modality_embedding_weight):
    seq_len = embeddings.shape[1]
    d = modality_embedding_weight.shape[1]
    rows_per_worker = seq_len // _NUM_WORKERS
    reps = rows_per_worker // _K

    idx = jnp.full((_KG,), token_type, dtype=jnp.int32)

    mesh = plsc.VectorSubcoreMesh(core_axis_name="c", subcore_axis_name="s")

    @functools.partial(
        pl.kernel,
        mesh=mesh,
        out_type=jax.ShapeDtypeStruct((seq_len, d), jnp.float32),
        scratch_types=[
            pltpu.VMEM((_KG,), jnp.int32),
            pltpu.VMEM((_K, d), jnp.float32),
            pltpu.SemaphoreType.DMA,
        ],
    )
    def bcast(idx_hbm, w_hbm, out_hbm, idx_v, buf_v, sem):
        wid = lax.axis_index("s") * _NUM_CORES + lax.axis_index("c")
        base = wid * rows_per_worker
        pltpu.sync_copy(idx_hbm, idx_v)
        pltpu.async_copy(w_hbm.at[idx_v], buf_v.at[pl.ds(0, _KG)], sem).wait()
        copies = [
            pltpu.async_copy(buf_v, out_hbm.at[pl.ds(base + j * _K, _K)], sem)
            for j in range(reps)
        ]
        for c in copies:
            c.wait()

    return bcast(idx, modality_embedding_weight)
